# baseline (speedup 1.0000x reference)
import functools

import jax
import jax.numpy as jnp
from jax import lax
from jax.experimental import pallas as pl
from jax.experimental.pallas import tpu as pltpu
from jax.experimental.pallas import tpu_sc as plsc


def _build(n_flat: int, dim: int, num_workers: int):
    per_w = n_flat // num_workers
    cb = 256
    n_chunks = per_w // cb
    assert per_w % cb == 0 and n_chunks % 2 == 0
    mesh = plsc.VectorSubcoreMesh(core_axis_name="c", subcore_axis_name="s")

    scratch = (
        [pltpu.VMEM((cb,), jnp.int32) for _ in range(2)]
        + [pltpu.VMEM((cb, dim), jnp.float32) for _ in range(2)]
        + [pltpu.SemaphoreType.DMA for _ in range(6)]
    )

    @functools.partial(
        pl.kernel,
        mesh=mesh,
        out_type=jax.ShapeDtypeStruct((n_flat, dim), jnp.float32),
        scratch_types=scratch,
        compiler_params=pltpu.CompilerParams(
            use_tc_tiling_on_sc=False, needs_layout_passes=False
        ),
    )
    def emb(w_hbm, x_hbm, out_hbm, *bufs):
        idx = bufs[0:2]
        rows = bufs[2:4]
        isem = bufs[4:6]
        gsem = bufs[6:8]
        ssem = bufs[8:10]
        nc = lax.axis_size("c")
        wid = lax.axis_index("s") * nc + lax.axis_index("c")
        base = wid * per_w

        def idx_load(i, s):
            pltpu.async_copy(x_hbm.at[pl.ds(base + i * cb, cb)], idx[s], isem[s])

        def idx_wait(s):
            pltpu.make_async_copy(x_hbm.at[pl.ds(0, cb)], idx[s], isem[s]).wait()

        def gather(i, s):
            pltpu.async_copy(w_hbm.at[idx[s]], rows[s], gsem[s])

        def gather_wait(s):
            pltpu.make_async_copy(w_hbm.at[pl.ds(0, cb)], rows[s], gsem[s]).wait()

        def store(i, s):
            pltpu.async_copy(rows[s], out_hbm.at[pl.ds(base + i * cb, cb)], ssem[s])

        def store_wait(s):
            pltpu.make_async_copy(
                rows[s], out_hbm.at[pl.ds(base, cb)], ssem[s]
            ).wait()

        def chunk_step(i, s, nxt, do_store_wait, load_next):
            if nxt is not None:
                idx_wait(1 - s)
                gather(nxt, 1 - s)
            gather_wait(s)
            if do_store_wait:
                store_wait(s)
            store(i, s)
            if load_next is not None:
                idx_load(load_next, s)

        idx_load(0, 0)
        idx_load(1, 1)
        idx_wait(0)
        gather(0, 0)
        chunk_step(0, 0, 1, False, 2)
        chunk_step(1, 1, 2, False, 3)

        def group(g, carry):
            i = 2 + 2 * g
            chunk_step(i, 0, i + 1, True, i + 2)
            chunk_step(i + 1, 1, i + 2, True, i + 3)
            return carry

        lax.fori_loop(0, (n_chunks - 4) // 2, group, 0)

        il = n_chunks - 2
        chunk_step(il, 0, il + 1, True, None)
        chunk_step(il + 1, 1, None, True, None)
        store_wait(0)
        store_wait(1)

    return emb


def kernel(x, weight):
    b, h = x.shape
    n_vocab, dim = weight.shape
    info = plsc.get_sparse_core_info()
    num_workers = info.num_cores * info.num_subcores
    emb = _build(b * h, dim, num_workers)
    o = emb(weight, x.reshape(b * h))
    return o.reshape(b, h, dim)


# speedup vs baseline: 1.0037x; 1.0037x over previous
import functools

import jax
import jax.numpy as jnp
from jax import lax
from jax.experimental import pallas as pl
from jax.experimental.pallas import tpu as pltpu
from jax.experimental.pallas import tpu_sc as plsc


def _build(n_flat: int, dim: int, num_workers: int):
    per_w = n_flat // num_workers
    cb = 512
    n_chunks = per_w // cb
    assert per_w % cb == 0 and n_chunks % 2 == 0
    mesh = plsc.VectorSubcoreMesh(core_axis_name="c", subcore_axis_name="s")

    scratch = (
        [pltpu.VMEM((cb,), jnp.int32) for _ in range(2)]
        + [pltpu.VMEM((cb, dim), jnp.float32) for _ in range(2)]
        + [pltpu.SemaphoreType.DMA for _ in range(6)]
    )

    @functools.partial(
        pl.kernel,
        mesh=mesh,
        out_type=jax.ShapeDtypeStruct((n_flat, dim), jnp.float32),
        scratch_types=scratch,
        compiler_params=pltpu.CompilerParams(
            use_tc_tiling_on_sc=False, needs_layout_passes=False
        ),
    )
    def emb(w_hbm, x_hbm, out_hbm, *bufs):
        idx = bufs[0:2]
        rows = bufs[2:4]
        isem = bufs[4:6]
        gsem = bufs[6:8]
        ssem = bufs[8:10]
        nc = lax.axis_size("c")
        wid = lax.axis_index("s") * nc + lax.axis_index("c")
        base = wid * per_w

        def idx_load(i, s):
            pltpu.async_copy(x_hbm.at[pl.ds(base + i * cb, cb)], idx[s], isem[s])

        def idx_wait(s):
            pltpu.make_async_copy(x_hbm.at[pl.ds(0, cb)], idx[s], isem[s]).wait()

        def gather(i, s):
            pltpu.async_copy(w_hbm.at[idx[s]], rows[s], gsem[s])

        def gather_wait(s):
            pltpu.make_async_copy(w_hbm.at[pl.ds(0, cb)], rows[s], gsem[s]).wait()

        def store(i, s):
            pltpu.async_copy(rows[s], out_hbm.at[pl.ds(base + i * cb, cb)], ssem[s])

        def store_wait(s):
            pltpu.make_async_copy(
                rows[s], out_hbm.at[pl.ds(base, cb)], ssem[s]
            ).wait()

        def chunk_step(i, s, nxt, do_store_wait, load_next):
            if nxt is not None:
                idx_wait(1 - s)
                gather(nxt, 1 - s)
            gather_wait(s)
            if do_store_wait:
                store_wait(s)
            store(i, s)
            if load_next is not None:
                idx_load(load_next, s)

        idx_load(0, 0)
        idx_load(1, 1)
        idx_wait(0)
        gather(0, 0)
        chunk_step(0, 0, 1, False, 2)
        chunk_step(1, 1, 2, False, 3)

        def group(g, carry):
            i = 2 + 2 * g
            chunk_step(i, 0, i + 1, True, i + 2)
            chunk_step(i + 1, 1, i + 2, True, i + 3)
            return carry

        lax.fori_loop(0, (n_chunks - 4) // 2, group, 0)

        il = n_chunks - 2
        chunk_step(il, 0, il + 1, True, None)
        chunk_step(il + 1, 1, None, True, None)
        store_wait(0)
        store_wait(1)

    return emb


def kernel(x, weight):
    b, h = x.shape
    n_vocab, dim = weight.shape
    info = plsc.get_sparse_core_info()
    num_workers = info.num_cores * info.num_subcores
    emb = _build(b * h, dim, num_workers)
    o = emb(weight, x.reshape(b * h))
    return o.reshape(b, h, dim)
